# baseline (speedup 1.0000x reference)
import functools

import jax
import jax.numpy as jnp
from jax import lax
from jax.experimental import pallas as pl
from jax.experimental.pallas import tpu as pltpu
from jax.experimental.pallas import tpu_sc as plsc

_B, _L, _D = 4096, 200, 64
_T = _B * _L
_NC, _NS = 2, 16
_NW = _NC * _NS
_TPW = _T // _NW
_SL = 128
_NSL = 4
_SB = _SL * _NSL
_NSB = _TPW // _SB
_N_AGE, _N_SEG, _N_POS = 120, 3, 256


def _embed_ln_sc(idx_d, idx_a, idx_s, idx_p,
                 diag_t, age_t, seg_t, pos_t, gamma, beta):
    mesh = plsc.VectorSubcoreMesh(core_axis_name="c", subcore_axis_name="s")

    @functools.partial(
        pl.kernel,
        out_type=jax.ShapeDtypeStruct((_T, _D), jnp.float32),
        mesh=mesh,
        scratch_types=[
            pltpu.VMEM((2, _NSL, _SL), jnp.int32),
            pltpu.VMEM((_SB,), jnp.int32),
            pltpu.VMEM((_SB,), jnp.int32),
            pltpu.VMEM((_SB,), jnp.int32),
            pltpu.VMEM((_N_AGE, _D), jnp.float32),
            pltpu.VMEM((_N_SEG, _D), jnp.float32),
            pltpu.VMEM((_N_POS, _D), jnp.float32),
            pltpu.VMEM((_D,), jnp.float32),
            pltpu.VMEM((_D,), jnp.float32),
            pltpu.VMEM((2, _SB, _D), jnp.float32),
            pltpu.SemaphoreType.DMA,
        ],
        compiler_params=pltpu.CompilerParams(
            use_tc_tiling_on_sc=False, needs_layout_passes=False),
    )
    def k(dt, at_, st, pt, id_, ia, is_, ip, g_hbm, b_hbm, out,
          vd, va, vs, vp, aget, segt, post, gv, bv, rows, sem):
        wid = lax.axis_index("s") * _NC + lax.axis_index("c")
        w0 = wid * _TPW
        w0r = w0 // _SL

        pltpu.sync_copy(at_, aget)
        pltpu.sync_copy(st, segt)
        pltpu.sync_copy(pt, post)
        pltpu.sync_copy(g_hbm, gv)
        pltpu.sync_copy(b_hbm, bv)

        def fire_gather(sb, buf):
            row0 = pl.multiple_of(w0r + sb * _NSL, _NSL)
            pltpu.sync_copy(id_.at[pl.ds(row0, _NSL)], vd.at[buf])
            for j in range(_NSL):
                pltpu.async_copy(
                    dt.at[vd.at[buf, j]],
                    rows.at[buf, pl.ds(j * _SL, _SL)], sem)

        def wait_gather(buf):
            for j in range(_NSL):
                pltpu.make_async_copy(
                    dt.at[vd.at[buf, j]],
                    rows.at[buf, pl.ds(j * _SL, _SL)], sem).wait()

        fire_gather(0, 0)

        cols = [lax.iota(jnp.int32, 16) + (16 * kk) for kk in range(4)]
        gk = [gv[pl.ds(16 * kk, 16)] for kk in range(4)]
        bk = [bv[pl.ds(16 * kk, 16)] for kk in range(4)]

        def do_block(sb, buf):
            wait_gather(buf)

            @pl.when(sb + 1 < _NSB)
            def _():
                fire_gather(sb + 1, 1 - buf)

            base = pl.multiple_of(w0 + sb * _SB, _SB)
            pltpu.sync_copy(ia.at[pl.ds(base, _SB)], va)
            pltpu.sync_copy(is_.at[pl.ds(base, _SB)], vs)
            pltpu.sync_copy(ip.at[pl.ds(base, _SB)], vp)

            @pl.loop(0, _SB // 16)
            def _grp(g):
                t0 = g * 16
                iva = va[pl.ds(t0, 16)]
                ivs = vs[pl.ds(t0, 16)]
                ivp = vp[pl.ds(t0, 16)]
                for i in range(16):
                    t = t0 + i
                    ai = jnp.full((16,), iva[i], dtype=jnp.int32)
                    si = jnp.full((16,), ivs[i], dtype=jnp.int32)
                    pi = jnp.full((16,), ivp[i], dtype=jnp.int32)
                    x = [
                        rows[buf, t, pl.ds(16 * kk, 16)]
                        + plsc.load_gather(aget, [ai, cols[kk]])
                        + plsc.load_gather(segt, [si, cols[kk]])
                        + plsc.load_gather(post, [pi, cols[kk]])
                        for kk in range(4)
                    ]
                    tot = jnp.sum(x[0] + x[1] + x[2] + x[3])
                    sq = [xx * xx for xx in x]
                    tot2 = jnp.sum(sq[0] + sq[1] + sq[2] + sq[3])
                    mean = jnp.full((16,), tot, dtype=jnp.float32) * (1.0 / 64.0)
                    m2 = jnp.full((16,), tot2, dtype=jnp.float32) * (1.0 / 64.0)
                    var = m2 - mean * mean + 1e-12
                    yi = jnp.int32(0x5F3759DF) - lax.shift_right_logical(
                        plsc.bitcast(var, jnp.int32), 1)
                    y = plsc.bitcast(yi, jnp.float32)
                    for _ in range(3):
                        y = y * (1.5 - 0.5 * var * y * y)
                    for kk in range(4):
                        rows[buf, t, pl.ds(16 * kk, 16)] = (
                            (x[kk] - mean) * y * gk[kk] + bk[kk])

            pltpu.sync_copy(rows.at[buf], out.at[pl.ds(base, _SB)])

        @pl.loop(0, _NSB // 2)
        def _pair(ii):
            do_block(ii * 2, 0)
            do_block(ii * 2 + 1, 1)

    return k(diag_t, age_t, seg_t, pos_t,
             idx_d, idx_a, idx_s, idx_p, gamma, beta)


def _mask_tc(pad):
    def body(p_ref, m_ref):
        m_ref[...] = p_ref[...] == 1

    bb = 256
    return pl.pallas_call(
        body,
        grid=(_B // bb,),
        in_specs=[pl.BlockSpec((bb, _L), lambda i: (i, 0))],
        out_specs=pl.BlockSpec((bb, _L), lambda i: (i, 0)),
        out_shape=jax.ShapeDtypeStruct((_B, _L), jnp.bool_),
    )(pad)


def kernel(diagnosis, age, segment, position, is_padding,
           diag_table, age_table, seg_table, pos_table, gamma, beta):
    emb = _embed_ln_sc(
        diagnosis.reshape(_T // _SL, _SL), age.reshape(-1),
        segment.reshape(-1), position.reshape(-1),
        diag_table, age_table, seg_table, pos_table, gamma, beta)
    return emb.reshape(_B, _L, _D), _mask_tc(is_padding)


# speedup vs baseline: 1.0321x; 1.0321x over previous
import functools

import jax
import jax.numpy as jnp
from jax import lax
from jax.experimental import pallas as pl
from jax.experimental.pallas import tpu as pltpu
from jax.experimental.pallas import tpu_sc as plsc

_B, _L, _D = 4096, 200, 64
_T = _B * _L
_NC, _NS = 2, 16
_NW = _NC * _NS
_TPW = _T // _NW
_SL = 128
_NSL = 4
_SB = _SL * _NSL
_NSB = _TPW // _SB
_N_AGE, _N_SEG, _N_POS = 120, 3, 256


def _embed_sum_sc(idx_d, idx_a, idx_s, idx_p,
                  diag_t, age_t, seg_t, pos_t):
    mesh = plsc.VectorSubcoreMesh(core_axis_name="c", subcore_axis_name="s")

    @functools.partial(
        pl.kernel,
        out_type=jax.ShapeDtypeStruct((_T, _D), jnp.float32),
        mesh=mesh,
        scratch_types=[
            pltpu.VMEM((2, _NSL, _SL), jnp.int32),
            pltpu.VMEM((_SB,), jnp.int32),
            pltpu.VMEM((_SB,), jnp.int32),
            pltpu.VMEM((_SB,), jnp.int32),
            pltpu.VMEM((_N_AGE, _D), jnp.float32),
            pltpu.VMEM((_N_SEG, _D), jnp.float32),
            pltpu.VMEM((_N_POS, _D), jnp.float32),
            pltpu.VMEM((2, _SB, _D), jnp.float32),
            pltpu.SemaphoreType.DMA,
        ],
        compiler_params=pltpu.CompilerParams(
            use_tc_tiling_on_sc=False, needs_layout_passes=False),
    )
    def k(dt, at_, st, pt, id_, ia, is_, ip, out,
          vd, va, vs, vp, aget, segt, post, rows, sem):
        wid = lax.axis_index("s") * _NC + lax.axis_index("c")
        w0 = wid * _TPW
        w0r = w0 // _SL

        pltpu.sync_copy(at_, aget)
        pltpu.sync_copy(st, segt)
        pltpu.sync_copy(pt, post)

        def fire_gather(sb, buf):
            row0 = pl.multiple_of(w0r + sb * _NSL, _NSL)
            pltpu.sync_copy(id_.at[pl.ds(row0, _NSL)], vd.at[buf])
            for j in range(_NSL):
                pltpu.async_copy(
                    dt.at[vd.at[buf, j]],
                    rows.at[buf, pl.ds(j * _SL, _SL)], sem)

        def wait_gather(buf):
            for j in range(_NSL):
                pltpu.make_async_copy(
                    dt.at[vd.at[buf, j]],
                    rows.at[buf, pl.ds(j * _SL, _SL)], sem).wait()

        fire_gather(0, 0)

        cols = [lax.iota(jnp.int32, 16) + (16 * kk) for kk in range(4)]

        def do_block(sb, buf):
            wait_gather(buf)

            @pl.when(sb + 1 < _NSB)
            def _():
                fire_gather(sb + 1, 1 - buf)

            base = pl.multiple_of(w0 + sb * _SB, _SB)
            pltpu.sync_copy(ia.at[pl.ds(base, _SB)], va)
            pltpu.sync_copy(is_.at[pl.ds(base, _SB)], vs)
            pltpu.sync_copy(ip.at[pl.ds(base, _SB)], vp)

            @pl.loop(0, _SB // 16)
            def _grp(g):
                t0 = g * 16
                iva = va[pl.ds(t0, 16)]
                ivs = vs[pl.ds(t0, 16)]
                ivp = vp[pl.ds(t0, 16)]
                for i in range(16):
                    t = t0 + i
                    ai = jnp.full((16,), iva[i], dtype=jnp.int32)
                    si = jnp.full((16,), ivs[i], dtype=jnp.int32)
                    pi = jnp.full((16,), ivp[i], dtype=jnp.int32)
                    for kk in range(4):
                        rows[buf, t, pl.ds(16 * kk, 16)] = (
                            rows[buf, t, pl.ds(16 * kk, 16)]
                            + plsc.load_gather(aget, [ai, cols[kk]])
                            + plsc.load_gather(segt, [si, cols[kk]])
                            + plsc.load_gather(post, [pi, cols[kk]]))

            pltpu.sync_copy(rows.at[buf], out.at[pl.ds(base, _SB)])

        @pl.loop(0, _NSB // 2)
        def _pair(ii):
            do_block(ii * 2, 0)
            do_block(ii * 2 + 1, 1)

    return k(diag_t, age_t, seg_t, pos_t,
             idx_d, idx_a, idx_s, idx_p)


def _ln_mask_tc(x, pad, gamma, beta):
    bb = 64

    def body(x_ref, p_ref, g_ref, b_ref, o_ref, m_ref):
        xv = x_ref[...].reshape(bb, _L, _D)
        mean = jnp.mean(xv, axis=-1, keepdims=True)
        cen = xv - mean
        var = jnp.mean(cen * cen, axis=-1, keepdims=True)
        o_ref[...] = cen * lax.rsqrt(var + 1e-12) * g_ref[...] + b_ref[...]
        m_ref[...] = p_ref[...] == 1

    return pl.pallas_call(
        body,
        grid=(_B // bb,),
        in_specs=[
            pl.BlockSpec((bb * _L, _D), lambda i: (i, 0)),
            pl.BlockSpec((bb, _L), lambda i: (i, 0)),
            pl.BlockSpec((_D,), lambda i: (0,)),
            pl.BlockSpec((_D,), lambda i: (0,)),
        ],
        out_specs=[
            pl.BlockSpec((bb, _L, _D), lambda i: (i, 0, 0)),
            pl.BlockSpec((bb, _L), lambda i: (i, 0)),
        ],
        out_shape=[
            jax.ShapeDtypeStruct((_B, _L, _D), jnp.float32),
            jax.ShapeDtypeStruct((_B, _L), jnp.bool_),
        ],
    )(x, pad, gamma, beta)


def kernel(diagnosis, age, segment, position, is_padding,
           diag_table, age_table, seg_table, pos_table, gamma, beta):
    summed = _embed_sum_sc(
        diagnosis.reshape(_T // _SL, _SL), age.reshape(-1),
        segment.reshape(-1), position.reshape(-1),
        diag_table, age_table, seg_table, pos_table)
    return _ln_mask_tc(summed, is_padding, gamma, beta)
